# baseline (speedup 1.0000x reference)
import functools

import jax
import jax.numpy as jnp
from jax import lax
from jax.experimental import pallas as pl
from jax.experimental.pallas import tpu as pltpu
from jax.experimental.pallas import tpu_sc as plsc

D_MODEL = 1024
D_HIDDEN = 1024
N_EXP = 16
N_TOK = 2048
BLK = 128
N_BLK = 32
PAD_N = N_BLK * BLK
N_WORKERS = 32
CHUNK = N_TOK // N_WORKERS


def _plan_kernel(x_ref, rw_ref, rb_ref, slot_ref, eob_ref, valid_ref,
                 xbi_ref):
    xx = x_ref[...]
    logits = jnp.dot(xx, rw_ref[...], preferred_element_type=jnp.float32)
    logits = logits + rb_ref[...]
    mx = jnp.max(logits, axis=1, keepdims=True)
    e_iota = lax.broadcasted_iota(jnp.int32, (N_TOK, N_EXP), 1)
    ids = jnp.min(jnp.where(logits == mx, e_iota, N_EXP), axis=1, keepdims=True)
    onehot = (e_iota == ids).astype(jnp.int32)

    csum = onehot
    sh = 1
    while sh < N_TOK:
        shifted = jnp.concatenate(
            [jnp.zeros((sh, N_EXP), jnp.int32), csum[: N_TOK - sh]], axis=0
        )
        csum = csum + shifted
        sh *= 2
    rank = jnp.sum(onehot * csum, axis=1, keepdims=True) - 1

    counts = jnp.sum(onehot, axis=0, keepdims=True)
    aligned = ((counts + BLK - 1) // BLK) * BLK
    acc = aligned
    sh = 1
    while sh < N_EXP:
        shifted = jnp.concatenate(
            [jnp.zeros((1, sh), jnp.int32), acc[:, : N_EXP - sh]], axis=1
        )
        acc = acc + shifted
        sh *= 2
    off = acc - aligned

    tok_off = jnp.sum(onehot * off, axis=1, keepdims=True)
    slot = tok_off + rank
    slot_ref[...] = jnp.broadcast_to(slot, (N_TOK, 128))

    total = jnp.sum(aligned, axis=1, keepdims=True)
    s0 = lax.broadcasted_iota(jnp.int32, (N_BLK, N_EXP), 0) * BLK
    s0 = jnp.minimum(s0, jnp.broadcast_to(total - BLK, (N_BLK, N_EXP)))
    eob = jnp.sum((jnp.broadcast_to(off, (N_BLK, N_EXP)) <= s0).astype(jnp.int32),
                  axis=1, keepdims=True) - 1
    b_iota = lax.broadcasted_iota(jnp.int32, (N_BLK, 1), 0)
    valid = b_iota * BLK < total
    xbi = jnp.minimum(b_iota, total // BLK - 1)
    eob_ref[...] = jnp.broadcast_to(eob, (N_BLK, 128))
    valid_ref[...] = jnp.broadcast_to(valid.astype(jnp.int32), (N_BLK, 128))
    xbi_ref[...] = jnp.broadcast_to(xbi, (N_BLK, 128))


_SC_MESH = plsc.VectorSubcoreMesh(core_axis_name="c", subcore_axis_name="s")

N_PIPE = 4
PIECE = CHUNK // N_PIPE


@functools.partial(
    pl.kernel,
    mesh=_SC_MESH,
    out_type=jax.ShapeDtypeStruct((PAD_N, D_MODEL), jnp.float32),
    scratch_types=[
        pltpu.VMEM((N_PIPE, PIECE), jnp.int32),
        pltpu.VMEM((CHUNK, D_MODEL), jnp.float32),
        pltpu.SemaphoreType.DMA,
        pltpu.SemaphoreType.DMA,
    ],
)
def _sc_scatter(x_hbm, slot_hbm, xpad_hbm, idx_v, rows_v, sem_in, sem_out):
    wid = lax.axis_index("s") * 2 + lax.axis_index("c")
    base = wid * CHUNK
    for p in range(N_PIPE):
        pltpu.sync_copy(slot_hbm.at[pl.ds(base + p * PIECE, PIECE)], idx_v.at[p])
    ins = [
        pltpu.async_copy(x_hbm.at[pl.ds(base + p * PIECE, PIECE)],
                         rows_v.at[pl.ds(p * PIECE, PIECE)], sem_in)
        for p in range(N_PIPE)
    ]
    outs = []
    for p in range(N_PIPE):
        ins[p].wait()
        outs.append(
            pltpu.async_copy(rows_v.at[pl.ds(p * PIECE, PIECE)],
                             xpad_hbm.at[idx_v.at[p]], sem_out))
    for c in outs:
        c.wait()


@functools.partial(
    pl.kernel,
    mesh=_SC_MESH,
    out_type=jax.ShapeDtypeStruct((N_TOK, D_HIDDEN), jnp.float32),
    scratch_types=[
        pltpu.VMEM((N_PIPE, PIECE), jnp.int32),
        pltpu.VMEM((CHUNK, D_HIDDEN), jnp.float32),
        pltpu.SemaphoreType.DMA,
        pltpu.SemaphoreType.DMA,
    ],
)
def _sc_gather(ypad_hbm, slot_hbm, out_hbm, idx_v, rows_v, sem_in, sem_out):
    wid = lax.axis_index("s") * 2 + lax.axis_index("c")
    base = wid * CHUNK
    for p in range(N_PIPE):
        pltpu.sync_copy(slot_hbm.at[pl.ds(base + p * PIECE, PIECE)], idx_v.at[p])
    ins = [
        pltpu.async_copy(ypad_hbm.at[idx_v.at[p]],
                         rows_v.at[pl.ds(p * PIECE, PIECE)], sem_in)
        for p in range(N_PIPE)
    ]
    outs = []
    for p in range(N_PIPE):
        ins[p].wait()
        outs.append(
            pltpu.async_copy(rows_v.at[pl.ds(p * PIECE, PIECE)],
                             out_hbm.at[pl.ds(base + p * PIECE, PIECE)], sem_out))
    for c in outs:
        c.wait()


def _moe_kernel(eob_sp, valid_sp, xbi_sp, xb_ref, w_ref, v_ref, y_ref):
    b = pl.program_id(0)

    @pl.when(valid_sp[b] == 1)
    def _body():
        xb = xb_ref[...]
        a = jnp.dot(xb, w_ref[0], preferred_element_type=jnp.float32)
        g = jnp.dot(xb, v_ref[0], preferred_element_type=jnp.float32)
        y_ref[...] = a * (g * (1.0 / (1.0 + jnp.exp(-g))))


def kernel(x, router_w, router_b, W, V):
    Bs, Ts, Dm = x.shape
    x_flat = x.reshape(Bs * Ts, Dm)
    rb = router_b.reshape(1, N_EXP)

    slot2d, eob2d, valid2d, xbi2d = pl.pallas_call(
        _plan_kernel,
        out_shape=[
            jax.ShapeDtypeStruct((N_TOK, 128), jnp.int32),
            jax.ShapeDtypeStruct((N_BLK, 128), jnp.int32),
            jax.ShapeDtypeStruct((N_BLK, 128), jnp.int32),
            jax.ShapeDtypeStruct((N_BLK, 128), jnp.int32),
        ],
    )(x_flat, router_w, rb)

    slot = slot2d[:, 0]
    eob = eob2d[:, 0]
    valid = valid2d[:, 0]
    xbi = xbi2d[:, 0]

    x_pad = _sc_scatter(x_flat, slot)

    grid_spec = pltpu.PrefetchScalarGridSpec(
        num_scalar_prefetch=3,
        grid=(N_BLK,),
        in_specs=[
            pl.BlockSpec((BLK, D_MODEL), lambda b, eob, val, xbi: (xbi[b], 0)),
            pl.BlockSpec((1, D_MODEL, D_HIDDEN),
                         lambda b, eob, val, xbi: (eob[b], 0, 0)),
            pl.BlockSpec((1, D_MODEL, D_HIDDEN),
                         lambda b, eob, val, xbi: (eob[b], 0, 0)),
        ],
        out_specs=pl.BlockSpec((BLK, D_HIDDEN),
                               lambda b, eob, val, xbi: (xbi[b], 0)),
    )
    y_pad = pl.pallas_call(
        _moe_kernel,
        grid_spec=grid_spec,
        out_shape=jax.ShapeDtypeStruct((PAD_N, D_HIDDEN), jnp.float32),
    )(eob, valid, xbi, x_pad, W, V)

    out = _sc_gather(y_pad, slot)
    return out.reshape(Bs, Ts, D_HIDDEN)


# speedup vs baseline: 1.0164x; 1.0164x over previous
import functools

import jax
import jax.numpy as jnp
from jax import lax
from jax.experimental import pallas as pl
from jax.experimental.pallas import tpu as pltpu
from jax.experimental.pallas import tpu_sc as plsc

D_MODEL = 1024
D_HIDDEN = 1024
N_EXP = 16
N_TOK = 2048
BLK = 128
N_BLK = 32
PAD_N = N_BLK * BLK
N_WORKERS = 32
CHUNK = N_TOK // N_WORKERS


def _plan_kernel(x_ref, rw_ref, rb_ref, slot_ref, eob_ref, valid_ref,
                 xbi_ref, lim_ref):
    xx = x_ref[...]
    logits = jnp.dot(xx, rw_ref[...], preferred_element_type=jnp.float32)
    logits = logits + rb_ref[...]
    mx = jnp.max(logits, axis=1, keepdims=True)
    e_iota = lax.broadcasted_iota(jnp.int32, (N_TOK, N_EXP), 1)
    ids = jnp.min(jnp.where(logits == mx, e_iota, N_EXP), axis=1, keepdims=True)
    onehot = (e_iota == ids).astype(jnp.int32)

    csum = onehot
    sh = 1
    while sh < N_TOK:
        shifted = jnp.concatenate(
            [jnp.zeros((sh, N_EXP), jnp.int32), csum[: N_TOK - sh]], axis=0
        )
        csum = csum + shifted
        sh *= 2
    rank = jnp.sum(onehot * csum, axis=1, keepdims=True) - 1

    counts = jnp.sum(onehot, axis=0, keepdims=True)
    aligned = ((counts + BLK - 1) // BLK) * BLK
    acc = aligned
    sh = 1
    while sh < N_EXP:
        shifted = jnp.concatenate(
            [jnp.zeros((1, sh), jnp.int32), acc[:, : N_EXP - sh]], axis=1
        )
        acc = acc + shifted
        sh *= 2
    off = acc - aligned

    tok_off = jnp.sum(onehot * off, axis=1, keepdims=True)
    slot = tok_off + rank
    slot_ref[...] = jnp.broadcast_to(slot, (N_TOK, 128))

    total = jnp.sum(aligned, axis=1, keepdims=True)
    s0 = lax.broadcasted_iota(jnp.int32, (N_BLK, N_EXP), 0) * BLK
    s0c = jnp.minimum(s0, jnp.broadcast_to(total - BLK, (N_BLK, N_EXP)))
    eob = jnp.sum((jnp.broadcast_to(off, (N_BLK, N_EXP)) <= s0c).astype(jnp.int32),
                  axis=1, keepdims=True) - 1
    b_iota = lax.broadcasted_iota(jnp.int32, (N_BLK, 1), 0)
    valid = b_iota * BLK < total
    xbi = jnp.minimum(b_iota, total // BLK - 1)
    e_iota_b = lax.broadcasted_iota(jnp.int32, (N_BLK, N_EXP), 1)
    eob_onehot = (e_iota_b == eob).astype(jnp.int32)
    gend = jnp.sum(eob_onehot * (off + counts), axis=1, keepdims=True)
    lim = jnp.clip(gend - b_iota * BLK, 0, BLK)
    eob_ref[...] = jnp.broadcast_to(eob, (N_BLK, 128))
    valid_ref[...] = jnp.broadcast_to(valid.astype(jnp.int32), (N_BLK, 128))
    xbi_ref[...] = jnp.broadcast_to(xbi, (N_BLK, 128))
    lim_ref[...] = jnp.broadcast_to(lim, (N_BLK, 128))


_SC_MESH = plsc.VectorSubcoreMesh(core_axis_name="c", subcore_axis_name="s")


@functools.partial(
    pl.kernel,
    mesh=_SC_MESH,
    out_type=jax.ShapeDtypeStruct((PAD_N, D_MODEL), jnp.float32),
    scratch_types=[
        pltpu.VMEM((CHUNK,), jnp.int32),
        pltpu.VMEM((CHUNK, D_MODEL), jnp.float32),
        pltpu.SemaphoreType.DMA,
    ],
)
def _sc_scatter(x_hbm, slot_hbm, xpad_hbm, idx_v, rows_v, sem):
    wid = lax.axis_index("s") * 2 + lax.axis_index("c")
    base = wid * CHUNK
    pltpu.sync_copy(slot_hbm.at[pl.ds(base, CHUNK)], idx_v)
    pltpu.sync_copy(x_hbm.at[pl.ds(base, CHUNK)], rows_v)
    pltpu.async_copy(rows_v, xpad_hbm.at[idx_v], sem).wait()


def _moe_kernel(eob_sp, valid_sp, xbi_sp, lim_sp, slot_ref, xb_ref, w_ref,
                v_ref, out_ref):
    b = pl.program_id(0)

    @pl.when(b == 0)
    def _init():
        out_ref[...] = jnp.zeros_like(out_ref)

    @pl.when(valid_sp[b] == 1)
    def _body():
        row = lax.broadcasted_iota(jnp.int32, (BLK, 1), 0)
        xb = jnp.where(row < lim_sp[b], xb_ref[...], 0.0)
        a = jnp.dot(xb, w_ref[0], preferred_element_type=jnp.float32)
        g = jnp.dot(xb, v_ref[0], preferred_element_type=jnp.float32)
        y = a * (g * (1.0 / (1.0 + jnp.exp(-g))))
        slot = slot_ref[:, 0:1]
        r = lax.broadcasted_iota(jnp.int32, (N_TOK, BLK), 1) + b * BLK
        pt = (slot == r).astype(jnp.float32)
        out_ref[...] += lax.dot_general(pt, y, (((1,), (0,)), ((), ())),
                                        preferred_element_type=jnp.float32)


def kernel(x, router_w, router_b, W, V):
    Bs, Ts, Dm = x.shape
    x_flat = x.reshape(Bs * Ts, Dm)
    rb = router_b.reshape(1, N_EXP)

    slot2d, eob2d, valid2d, xbi2d, lim2d = pl.pallas_call(
        _plan_kernel,
        out_shape=[
            jax.ShapeDtypeStruct((N_TOK, 128), jnp.int32),
            jax.ShapeDtypeStruct((N_BLK, 128), jnp.int32),
            jax.ShapeDtypeStruct((N_BLK, 128), jnp.int32),
            jax.ShapeDtypeStruct((N_BLK, 128), jnp.int32),
            jax.ShapeDtypeStruct((N_BLK, 128), jnp.int32),
        ],
    )(x_flat, router_w, rb)

    slot = slot2d[:, 0]
    eob = eob2d[:, 0]
    valid = valid2d[:, 0]
    xbi = xbi2d[:, 0]
    lim = lim2d[:, 0]

    x_pad = _sc_scatter(x_flat, slot)

    grid_spec = pltpu.PrefetchScalarGridSpec(
        num_scalar_prefetch=4,
        grid=(N_BLK,),
        in_specs=[
            pl.BlockSpec((N_TOK, 128), lambda b, eob, val, xbi, lim: (0, 0)),
            pl.BlockSpec((BLK, D_MODEL), lambda b, eob, val, xbi, lim: (xbi[b], 0)),
            pl.BlockSpec((1, D_MODEL, D_HIDDEN),
                         lambda b, eob, val, xbi, lim: (eob[b], 0, 0)),
            pl.BlockSpec((1, D_MODEL, D_HIDDEN),
                         lambda b, eob, val, xbi, lim: (eob[b], 0, 0)),
        ],
        out_specs=pl.BlockSpec((N_TOK, D_HIDDEN),
                               lambda b, eob, val, xbi, lim: (0, 0)),
    )
    out = pl.pallas_call(
        _moe_kernel,
        grid_spec=grid_spec,
        out_shape=jax.ShapeDtypeStruct((N_TOK, D_HIDDEN), jnp.float32),
    )(eob, valid, xbi, lim, slot2d, x_pad, W, V)
    return out.reshape(Bs, Ts, D_HIDDEN)


# speedup vs baseline: 1.0344x; 1.0178x over previous
import functools

import jax
import jax.numpy as jnp
from jax import lax
from jax.experimental import pallas as pl
from jax.experimental.pallas import tpu as pltpu
from jax.experimental.pallas import tpu_sc as plsc

D_MODEL = 1024
D_HIDDEN = 1024
N_EXP = 16
N_TOK = 2048
BLK = 128
N_BLK = 32
PAD_N = N_BLK * BLK
N_WORKERS = 32
CHUNK = N_TOK // N_WORKERS


def _plan_kernel(x_ref, rw_ref, rb_ref, slot_ref, eob_ref, valid_ref,
                 xbi_ref, lim_ref):
    xx = x_ref[...]
    logits = jnp.dot(xx, rw_ref[...], preferred_element_type=jnp.float32)
    logits = logits + rb_ref[...]
    mx = jnp.max(logits, axis=1, keepdims=True)
    e_iota = lax.broadcasted_iota(jnp.int32, (N_TOK, N_EXP), 1)
    ids = jnp.min(jnp.where(logits == mx, e_iota, N_EXP), axis=1, keepdims=True)
    onehot = (e_iota == ids).astype(jnp.int32)

    csum = onehot
    sh = 1
    while sh < N_TOK:
        shifted = jnp.concatenate(
            [jnp.zeros((sh, N_EXP), jnp.int32), csum[: N_TOK - sh]], axis=0
        )
        csum = csum + shifted
        sh *= 2
    rank = jnp.sum(onehot * csum, axis=1, keepdims=True) - 1

    counts = jnp.sum(onehot, axis=0, keepdims=True)
    aligned = ((counts + BLK - 1) // BLK) * BLK
    acc = aligned
    sh = 1
    while sh < N_EXP:
        shifted = jnp.concatenate(
            [jnp.zeros((1, sh), jnp.int32), acc[:, : N_EXP - sh]], axis=1
        )
        acc = acc + shifted
        sh *= 2
    off = acc - aligned

    tok_off = jnp.sum(onehot * off, axis=1, keepdims=True)
    slot = tok_off + rank
    slot_ref[...] = jnp.broadcast_to(slot, (N_TOK, 128))

    total = jnp.sum(aligned, axis=1, keepdims=True)
    s0 = lax.broadcasted_iota(jnp.int32, (N_BLK, N_EXP), 0) * BLK
    s0c = jnp.minimum(s0, jnp.broadcast_to(total - BLK, (N_BLK, N_EXP)))
    eob = jnp.sum((jnp.broadcast_to(off, (N_BLK, N_EXP)) <= s0c).astype(jnp.int32),
                  axis=1, keepdims=True) - 1
    b_iota = lax.broadcasted_iota(jnp.int32, (N_BLK, 1), 0)
    valid = b_iota * BLK < total
    xbi = jnp.minimum(b_iota, total // BLK - 1)
    e_iota_b = lax.broadcasted_iota(jnp.int32, (N_BLK, N_EXP), 1)
    eob_onehot = (e_iota_b == eob).astype(jnp.int32)
    gend = jnp.sum(eob_onehot * (off + counts), axis=1, keepdims=True)
    lim = jnp.clip(gend - b_iota * BLK, 0, BLK)
    eob_ref[...] = jnp.broadcast_to(eob, (N_BLK, 128))
    valid_ref[...] = jnp.broadcast_to(valid.astype(jnp.int32), (N_BLK, 128))
    xbi_ref[...] = jnp.broadcast_to(xbi, (N_BLK, 128))
    lim_ref[...] = jnp.broadcast_to(lim, (N_BLK, 128))


@functools.lru_cache(maxsize=1)
def _make_sc_scatter():
    mesh = plsc.VectorSubcoreMesh(core_axis_name="c", subcore_axis_name="s")

    @functools.partial(
        pl.kernel,
        mesh=mesh,
        out_type=jax.ShapeDtypeStruct((PAD_N, D_MODEL), jnp.float32),
        scratch_types=[
            pltpu.VMEM((CHUNK,), jnp.int32),
            pltpu.VMEM((CHUNK, D_MODEL), jnp.float32),
            pltpu.SemaphoreType.DMA,
        ],
    )
    def _sc_scatter(x_hbm, slot_hbm, xpad_hbm, idx_v, rows_v, sem):
        wid = lax.axis_index("s") * 2 + lax.axis_index("c")
        base = wid * CHUNK
        pltpu.sync_copy(slot_hbm.at[pl.ds(base, CHUNK)], idx_v)
        pltpu.sync_copy(x_hbm.at[pl.ds(base, CHUNK)], rows_v)
        pltpu.async_copy(rows_v, xpad_hbm.at[idx_v], sem).wait()

    return _sc_scatter


def _moe_kernel(eob_sp, valid_sp, xbi_sp, lim_sp, slot_ref, xb_ref, w_ref,
                v_ref, out_ref):
    b = pl.program_id(0)

    @pl.when(b == 0)
    def _init():
        out_ref[...] = jnp.zeros_like(out_ref)

    @pl.when(valid_sp[b] == 1)
    def _body():
        row = lax.broadcasted_iota(jnp.int32, (BLK, 1), 0)
        xb = jnp.where(row < lim_sp[b], xb_ref[...], 0.0)
        a = jnp.dot(xb, w_ref[0], preferred_element_type=jnp.float32)
        g = jnp.dot(xb, v_ref[0], preferred_element_type=jnp.float32)
        y = a * (g * (1.0 / (1.0 + jnp.exp(-g))))
        slot = slot_ref[:, 0:1]
        r = lax.broadcasted_iota(jnp.int32, (N_TOK, BLK), 1) + b * BLK
        pt = (slot == r).astype(jnp.float32)
        out_ref[...] += lax.dot_general(pt, y, (((1,), (0,)), ((), ())),
                                        preferred_element_type=jnp.float32)


def kernel(x, router_w, router_b, W, V):
    Bs, Ts, Dm = x.shape
    x_flat = x.reshape(Bs * Ts, Dm)
    rb = router_b.reshape(1, N_EXP)

    slot2d, eob2d, valid2d, xbi2d, lim2d = pl.pallas_call(
        _plan_kernel,
        out_shape=[
            jax.ShapeDtypeStruct((N_TOK, 128), jnp.int32),
            jax.ShapeDtypeStruct((N_BLK, 128), jnp.int32),
            jax.ShapeDtypeStruct((N_BLK, 128), jnp.int32),
            jax.ShapeDtypeStruct((N_BLK, 128), jnp.int32),
            jax.ShapeDtypeStruct((N_BLK, 128), jnp.int32),
        ],
    )(x_flat, router_w, rb)

    slot = slot2d[:, 0]
    eob = eob2d[:, 0]
    valid = valid2d[:, 0]
    xbi = xbi2d[:, 0]
    lim = lim2d[:, 0]

    x_pad = _make_sc_scatter()(x_flat, slot)

    grid_spec = pltpu.PrefetchScalarGridSpec(
        num_scalar_prefetch=4,
        grid=(N_BLK,),
        in_specs=[
            pl.BlockSpec((N_TOK, 128), lambda b, eob, val, xbi, lim: (0, 0)),
            pl.BlockSpec((BLK, D_MODEL), lambda b, eob, val, xbi, lim: (xbi[b], 0)),
            pl.BlockSpec((1, D_MODEL, D_HIDDEN),
                         lambda b, eob, val, xbi, lim: (eob[b], 0, 0)),
            pl.BlockSpec((1, D_MODEL, D_HIDDEN),
                         lambda b, eob, val, xbi, lim: (eob[b], 0, 0)),
        ],
        out_specs=pl.BlockSpec((N_TOK, D_HIDDEN),
                               lambda b, eob, val, xbi, lim: (0, 0)),
    )
    out = pl.pallas_call(
        _moe_kernel,
        grid_spec=grid_spec,
        out_shape=jax.ShapeDtypeStruct((N_TOK, D_HIDDEN), jnp.float32),
    )(eob, valid, xbi, lim, slot2d, x_pad, W, V)
    return out.reshape(Bs, Ts, D_HIDDEN)
